# baseline (speedup 1.0000x reference)
import functools

import jax
import jax.numpy as jnp
from jax import lax
from jax.experimental import pallas as pl
from jax.experimental.pallas import tpu as pltpu
from jax.experimental.pallas import tpu_sc as plsc

GAMMA = 24.0
LANES = 16


def kernel(entity_table, relation_table, positive_sample, negative_sample,
           q_entity, q_relation):
    B = positive_sample.shape[0]
    NNEG = negative_sample.shape[1]
    D = entity_table.shape[1]
    VPR = D // LANES

    info = plsc.get_sparse_core_info()
    NC, NS = info.num_cores, info.num_subcores
    NW = NC * NS
    BPW = B // NW

    mesh = plsc.VectorSubcoreMesh(core_axis_name="c", subcore_axis_name="s")

    @functools.partial(
        pl.kernel,
        out_type=(
            jax.ShapeDtypeStruct((B,), jnp.float32),
            jax.ShapeDtypeStruct((B, NNEG), jnp.float32),
        ),
        mesh=mesh,
        compiler_params=pltpu.CompilerParams(use_tc_tiling_on_sc=False),
        scratch_types=[
            pltpu.VMEM((BPW,), jnp.int32),
            pltpu.VMEM((BPW,), jnp.int32),
            pltpu.VMEM((BPW,), jnp.int32),
            pltpu.VMEM((BPW, NNEG), jnp.int32),
            pltpu.VMEM((BPW, D), jnp.float32),
            pltpu.VMEM((BPW, D), jnp.float32),
            pltpu.VMEM((2, NNEG, D), jnp.float32),
            pltpu.VMEM((BPW,), jnp.float32),
            pltpu.VMEM((BPW, NNEG), jnp.float32),
            pltpu.SemaphoreType.DMA,
            pltpu.SemaphoreType.DMA,
        ],
    )
    def gqe_kernel(ent_hbm, rel_hbm, pos_hbm, neg_hbm, qe_hbm, qr_hbm,
                   pos_out, neg_out,
                   qe_idx, qr_idx, pos_idx, neg_idx, q_rows, tmp_rows,
                   neg_buf, pos_stage, neg_stage, sem0, sem1):
        wid = lax.axis_index("s") * NC + lax.axis_index("c")
        base = wid * BPW

        pltpu.sync_copy(qe_hbm.at[pl.ds(base, BPW)], qe_idx)
        pltpu.sync_copy(qr_hbm.at[pl.ds(base, BPW)], qr_idx)
        pltpu.sync_copy(pos_hbm.at[pl.ds(base, BPW)], pos_idx)
        pltpu.sync_copy(neg_hbm.at[pl.ds(base, BPW)], neg_idx)

        cp_q = pltpu.async_copy(ent_hbm.at[qe_idx], q_rows, sem0)
        cp_r = pltpu.async_copy(rel_hbm.at[qr_idx], tmp_rows, sem1)
        cp_q.wait()
        cp_r.wait()

        def qbody(r, _):
            for c in range(VPR):
                q_rows[r, pl.ds(c * LANES, LANES)] = (
                    q_rows[r, pl.ds(c * LANES, LANES)]
                    + tmp_rows[r, pl.ds(c * LANES, LANES)])
            return 0
        lax.fori_loop(0, BPW, qbody, 0)

        pltpu.async_copy(ent_hbm.at[pos_idx], tmp_rows, sem0).wait()

        lane = lax.iota(jnp.int32, LANES)

        def hsum_splat(v):
            for sft in (8, 4, 2, 1):
                v = v + jnp.take_along_axis(v, lane ^ sft, axis=0)
            return v

        def pbody(g, _):
            res = jnp.zeros((LANES,), jnp.float32)
            for j in range(LANES):
                r = g * LANES + j
                acc = jnp.zeros((LANES,), jnp.float32)
                for c in range(VPR):
                    qv = q_rows[r, pl.ds(c * LANES, LANES)]
                    pv = tmp_rows[r, pl.ds(c * LANES, LANES)]
                    acc = acc + jnp.abs(pv - qv)
                res = jnp.where(lane == j, GAMMA - hsum_splat(acc), res)
            pos_stage[pl.ds(g * LANES, LANES)] = res
            return 0
        lax.fori_loop(0, BPW // LANES, pbody, 0)

        pltpu.async_copy(ent_hbm.at[neg_idx.at[0]], neg_buf.at[0], sem0)
        pltpu.async_copy(ent_hbm.at[neg_idx.at[1]], neg_buf.at[1], sem1)

        def nouter(bb, _):
            for phase in range(2):
                b = bb * 2 + phase
                buf = neg_buf.at[phase]
                sem = sem0 if phase == 0 else sem1
                pltpu.make_async_copy(ent_hbm.at[neg_idx.at[b]], buf,
                                      sem).wait()
                qv = [q_rows[b, pl.ds(c * LANES, LANES)] for c in range(VPR)]

                def kbody(kk, _, buf=buf, qv=qv, b=b):
                    res = jnp.zeros((LANES,), jnp.float32)
                    for j in range(LANES):
                        k = kk * LANES + j
                        acc = None
                        for c in range(VPR):
                            nv = buf[k, pl.ds(c * LANES, LANES)]
                            a = jnp.abs(nv - qv[c])
                            acc = a if acc is None else acc + a
                        res = jnp.where(lane == j, GAMMA - hsum_splat(acc),
                                        res)
                    neg_stage[b, pl.ds(kk * LANES, LANES)] = res
                    return 0
                lax.fori_loop(0, NNEG // LANES, kbody, 0)

                @pl.when(b + 2 < BPW)
                def _():
                    pltpu.async_copy(ent_hbm.at[neg_idx.at[b + 2]], buf, sem)
            return 0
        lax.fori_loop(0, BPW // 2, nouter, 0)

        pltpu.sync_copy(pos_stage, pos_out.at[pl.ds(base, BPW)])
        pltpu.sync_copy(neg_stage, neg_out.at[pl.ds(base, BPW)])

    return gqe_kernel(entity_table, relation_table, positive_sample,
                      negative_sample, q_entity, q_relation)
